# baseline (speedup 1.0000x reference)
import functools

import jax
import jax.numpy as jnp
from jax import lax
from jax.experimental import pallas as pl
from jax.experimental.pallas import tpu as pltpu
from jax.experimental.pallas import tpu_sc as plsc

N_FACTORS = 64
EXT = N_FACTORS + 16
BATCH = 16384
N_ROWS = 1000000

_info = plsc.get_sparse_core_info()
_NC, _NS, _L = _info.num_cores, _info.num_subcores, _info.num_lanes
_NW = _NC * _NS
_BPW = BATCH // _NW
_RING = 3
_NBLK = (N_ROWS + 127) // 128
_LASTW = N_ROWS - (_NBLK - 1) * 128


def _extract_body(is_user, blk_hbm, col_hbm, perm_hbm, ft_hbm, bias_hbm,
                  out_hbm, blk_v, col_v, perm_v, rowbuf_v,
                  slabs, bslabs, tail_v, btail_v, sem, sem_out):
    wid = lax.axis_index("s") * _NC + lax.axis_index("c")
    base = wid * _BPW
    lanes = lax.iota(jnp.int32, _L)
    zeros = lanes * 0
    ones_f = (zeros + 1).astype(jnp.float32)
    zeros_f = zeros.astype(jnp.float32)

    pltpu.sync_copy(blk_hbm.at[pl.ds(base, _BPW)], blk_v)
    pltpu.sync_copy(col_hbm.at[pl.ds(base, _BPW)], col_v)
    pltpu.sync_copy(perm_hbm.at[pl.ds(base, _BPW)], perm_v)

    b_lo = blk_v[pl.ds(0, _L)][0]
    b_hi = blk_v[pl.ds(_BPW - _L, _L)][_L - 1]

    def slab_copies(s, b):
        off = pl.multiple_of(b * 128, 128)
        full = pltpu.make_async_copy(
            ft_hbm.at[pl.ds(0, N_FACTORS), pl.ds(off, 128)], slabs[s], sem)
        bfull = pltpu.make_async_copy(
            bias_hbm.at[pl.ds(0, 1), pl.ds(off, 128)], bslabs[s], sem)
        tailoff = (_NBLK - 1) * 128
        tail = pltpu.make_async_copy(
            ft_hbm.at[pl.ds(0, N_FACTORS), pl.ds(tailoff, _LASTW)],
            tail_v, sem)
        btail = pltpu.make_async_copy(
            bias_hbm.at[pl.ds(0, 1), pl.ds(tailoff, _LASTW)], btail_v, sem)
        return full, bfull, tail, btail

    def fire(s, b):
        full, bfull, tail, btail = slab_copies(s, b)
        @pl.when(b < _NBLK - 1)
        def _():
            full.start(); bfull.start()
        @pl.when(b == _NBLK - 1)
        def _():
            tail.start(); btail.start()

    def drain(s, b):
        full, bfull, tail, btail = slab_copies(s, b)
        @pl.when((b <= b_hi) & (b < _NBLK - 1))
        def _():
            full.wait(); bfull.wait()
        @pl.when((b <= b_hi) & (b == _NBLK - 1))
        def _():
            tail.wait(); btail.wait()

    for s in range(_RING):
        @pl.when(b_lo + s <= b_hi)
        def _(s=s):
            fire(s, b_lo + s)

    def blk_at(k):
        v = plsc.load_gather(blk_v, [zeros + jnp.minimum(k, _BPW - 1)])[0]
        return jnp.where(k < _BPW, v, jnp.int32(-1))

    def ring_step(j, carry):
        k = carry
        for s in range(_RING):
            b = b_lo + j * _RING + s
            drain(s, b)

            def ext_cond(k2):
                return blk_at(k2) == b

            def ext_body(k2):
                c = plsc.load_gather(col_v, [zeros + k2])[0]
                ct = jnp.minimum(c, _LASTW - 1)
                is_tail = b == _NBLK - 1
                for q in range(N_FACTORS // _L):
                    vec = plsc.load_gather(slabs[s], [q * _L + lanes, zeros + c])
                    tvec = plsc.load_gather(tail_v, [q * _L + lanes, zeros + ct])
                    rowbuf_v[k2, pl.ds(q * _L, _L)] = jnp.where(is_tail, tvec, vec)
                bv = plsc.load_gather(bslabs[s], [zeros, zeros + c])
                btv = plsc.load_gather(btail_v, [zeros, zeros + ct])
                bval = jnp.where(is_tail, btv, bv)
                if is_user:
                    slot = jnp.where(lanes == 0, bval,
                                     jnp.where(lanes == 1, ones_f, zeros_f))
                else:
                    slot = jnp.where(lanes == 1, bval,
                                     jnp.where(lanes == 0, ones_f, zeros_f))
                rowbuf_v[k2, pl.ds(N_FACTORS, _L)] = slot
                return k2 + 1

            k = lax.while_loop(ext_cond, ext_body, k)
            bn = b + _RING
            @pl.when(bn <= b_hi)
            def _(s=s, bn=bn):
                fire(s, bn)
        return k

    nsteps = (b_hi - b_lo + _RING) // _RING
    lax.fori_loop(0, nsteps, ring_step, jnp.int32(0))

    def scatter_block(g, _):
        k0 = g * _L
        pvec = perm_v[pl.ds(k0, _L)]
        for r in range(_L):
            pltpu.make_async_copy(rowbuf_v.at[k0 + r],
                                  out_hbm.at[pvec[r]], sem_out).start()
        return 0

    lax.fori_loop(0, _BPW // _L, scatter_block, 0)
    pltpu.make_async_copy(out_hbm.at[pl.ds(0, _BPW)], rowbuf_v, sem_out).wait()


def _gather_a_body(ublk_hbm, ucol_hbm, pu_hbm, mblk_hbm, mcol_hbm, pm_hbm,
                   uft_hbm, mft_hbm, ub_hbm, mb_hbm,
                   urows_hbm, mrows_hbm,
                   blk_v, col_v, perm_v, rowbuf_v,
                   s0, s1, s2, b0, b1, b2,
                   tail_v, btail_v, sem, sem_out):
    slabs = (s0, s1, s2)
    bslabs = (b0, b1, b2)
    _extract_body(True, ublk_hbm, ucol_hbm, pu_hbm, uft_hbm, ub_hbm,
                  urows_hbm, blk_v, col_v, perm_v, rowbuf_v,
                  slabs, bslabs, tail_v, btail_v, sem, sem_out)
    _extract_body(False, mblk_hbm, mcol_hbm, pm_hbm, mft_hbm, mb_hbm,
                  mrows_hbm, blk_v, col_v, perm_v, rowbuf_v,
                  slabs, bslabs, tail_v, btail_v, sem, sem_out)


def _dot_body(urows_hbm, mrows_hbm, out_hbm,
              urows_v, mrows_v, out_v, pacc_v, sem_u, sem_m):
    wid = lax.axis_index("s") * _NC + lax.axis_index("c")
    base = wid * _BPW

    lanes = lax.iota(jnp.int32, _L)
    half = _BPW // 2

    def chunk(h, _):
        c0 = h * half
        cu = pltpu.make_async_copy(
            urows_hbm.at[pl.ds(base + c0, half)], urows_v, sem_u)
        cm = pltpu.make_async_copy(
            mrows_hbm.at[pl.ds(base + c0, half)], mrows_v, sem_m)
        cu.start(); cm.start(); cu.wait(); cm.wait()

        def block(b, _):
            r0 = b * _L
            for r in range(_L):
                acc = urows_v[r0 + r, pl.ds(0, _L)] * mrows_v[r0 + r, pl.ds(0, _L)]
                for j in range(1, EXT // _L):
                    acc = acc + (urows_v[r0 + r, pl.ds(j * _L, _L)]
                                 * mrows_v[r0 + r, pl.ds(j * _L, _L)])
                pacc_v[pl.ds(r * _L, _L)] = acc
            tot = plsc.load_gather(pacc_v, [lanes * _L])
            for l in range(1, _L):
                tot = tot + plsc.load_gather(pacc_v, [lanes * _L + l])
            out_v[pl.ds(c0 + r0, _L)] = tot
            return 0

        lax.fori_loop(0, half // _L, block, 0)
        return 0

    lax.fori_loop(0, 2, chunk, 0)
    pltpu.sync_copy(out_v, out_hbm.at[pl.ds(base, _BPW)])


@jax.jit
def kernel(user, movie, user_factors, movie_factors, user_biases, movie_biases):
    mesh = plsc.VectorSubcoreMesh(core_axis_name="c", subcore_axis_name="s")

    run_a = pl.kernel(
        _gather_a_body,
        out_type=(jax.ShapeDtypeStruct((BATCH, EXT), jnp.float32),
                  jax.ShapeDtypeStruct((BATCH, EXT), jnp.float32)),
        mesh=mesh,
        compiler_params=pltpu.CompilerParams(
            needs_layout_passes=False, use_tc_tiling_on_sc=True),
        scratch_types=[
            pltpu.VMEM((_BPW,), jnp.int32),
            pltpu.VMEM((_BPW,), jnp.int32),
            pltpu.VMEM((_BPW,), jnp.int32),
            pltpu.VMEM((_BPW, EXT), jnp.float32),
            pltpu.VMEM((N_FACTORS, 128), jnp.float32),
            pltpu.VMEM((N_FACTORS, 128), jnp.float32),
            pltpu.VMEM((N_FACTORS, 128), jnp.float32),
            pltpu.VMEM((1, 128), jnp.float32),
            pltpu.VMEM((1, 128), jnp.float32),
            pltpu.VMEM((1, 128), jnp.float32),
            pltpu.VMEM((N_FACTORS, _LASTW), jnp.float32),
            pltpu.VMEM((1, _LASTW), jnp.float32),
            pltpu.SemaphoreType.DMA,
            pltpu.SemaphoreType.DMA,
        ],
    )

    run_b = pl.kernel(
        _dot_body,
        out_type=jax.ShapeDtypeStruct((BATCH,), jnp.float32),
        mesh=mesh,
        compiler_params=pltpu.CompilerParams(
            needs_layout_passes=False, use_tc_tiling_on_sc=True),
        scratch_types=[
            pltpu.VMEM((_BPW // 2, EXT), jnp.float32),
            pltpu.VMEM((_BPW // 2, EXT), jnp.float32),
            pltpu.VMEM((_BPW,), jnp.float32),
            pltpu.VMEM((_L * _L,), jnp.float32),
            pltpu.SemaphoreType.DMA,
            pltpu.SemaphoreType.DMA,
        ],
    )

    pu = jnp.argsort(user).astype(jnp.int32)
    pm = jnp.argsort(movie).astype(jnp.int32)
    su = jnp.take(user, pu)
    sm = jnp.take(movie, pm)

    urows, mrows = run_a(su >> 7, su & 127, pu, sm >> 7, sm & 127, pm,
                         user_factors.T, movie_factors.T,
                         user_biases.T, movie_biases.T)
    return run_b(urows, mrows)


# speedup vs baseline: 1.1323x; 1.1323x over previous
import functools

import jax
import jax.numpy as jnp
from jax import lax
from jax.experimental import pallas as pl
from jax.experimental.pallas import tpu as pltpu
from jax.experimental.pallas import tpu_sc as plsc

N_FACTORS = 64
EXT = N_FACTORS + 16
BATCH = 16384
N_ROWS = 1000000

_info = plsc.get_sparse_core_info()
_NC, _NS, _L = _info.num_cores, _info.num_subcores, _info.num_lanes
_NW = _NC * _NS
_BPW = BATCH // _NW
_RING = 4
_NBLK = (N_ROWS + 127) // 128
_LASTW = N_ROWS - (_NBLK - 1) * 128


def _extract_body(is_user, blk_hbm, col_hbm, perm_hbm, ft_hbm, bias_hbm,
                  out_hbm, blk_v, col_v, perm_v, rowbuf_v,
                  slabs, bslabs, tail_v, btail_v, sem, sem_out):
    wid = lax.axis_index("s") * _NC + lax.axis_index("c")
    base = wid * _BPW
    lanes = lax.iota(jnp.int32, _L)
    zeros = lanes * 0
    ones_f = (zeros + 1).astype(jnp.float32)
    zeros_f = zeros.astype(jnp.float32)

    pltpu.sync_copy(blk_hbm.at[pl.ds(base, _BPW)], blk_v)
    pltpu.sync_copy(col_hbm.at[pl.ds(base, _BPW)], col_v)
    pltpu.sync_copy(perm_hbm.at[pl.ds(base, _BPW)], perm_v)

    b_lo = blk_v[pl.ds(0, _L)][0]
    b_hi = blk_v[pl.ds(_BPW - _L, _L)][_L - 1]

    def slab_copies(s, b):
        off = pl.multiple_of(b * 128, 128)
        full = pltpu.make_async_copy(
            ft_hbm.at[pl.ds(0, N_FACTORS), pl.ds(off, 128)], slabs[s], sem)
        bfull = pltpu.make_async_copy(
            bias_hbm.at[pl.ds(0, 1), pl.ds(off, 128)], bslabs[s], sem)
        tailoff = (_NBLK - 1) * 128
        tail = pltpu.make_async_copy(
            ft_hbm.at[pl.ds(0, N_FACTORS), pl.ds(tailoff, _LASTW)],
            tail_v, sem)
        btail = pltpu.make_async_copy(
            bias_hbm.at[pl.ds(0, 1), pl.ds(tailoff, _LASTW)], btail_v, sem)
        return full, bfull, tail, btail

    def fire(s, b):
        full, bfull, tail, btail = slab_copies(s, b)
        @pl.when(b < _NBLK - 1)
        def _():
            full.start(); bfull.start()
        @pl.when(b == _NBLK - 1)
        def _():
            tail.start(); btail.start()

    def drain(s, b):
        full, bfull, tail, btail = slab_copies(s, b)
        @pl.when((b <= b_hi) & (b < _NBLK - 1))
        def _():
            full.wait(); bfull.wait()
        @pl.when((b <= b_hi) & (b == _NBLK - 1))
        def _():
            tail.wait(); btail.wait()

    for s in range(_RING):
        @pl.when(b_lo + s <= b_hi)
        def _(s=s):
            fire(s, b_lo + s)

    def blk_at(k):
        v = plsc.load_gather(blk_v, [zeros + jnp.minimum(k, _BPW - 1)])[0]
        return jnp.where(k < _BPW, v, jnp.int32(-1))

    def ring_step(j, carry):
        k = carry
        for s in range(_RING):
            b = b_lo + j * _RING + s
            drain(s, b)

            def ext_cond(k2):
                return blk_at(k2) == b

            def ext_body(k2):
                c = plsc.load_gather(col_v, [zeros + k2])[0]
                ct = jnp.minimum(c, _LASTW - 1)
                is_tail = b == _NBLK - 1
                for q in range(N_FACTORS // _L):
                    vec = plsc.load_gather(slabs[s], [q * _L + lanes, zeros + c])
                    tvec = plsc.load_gather(tail_v, [q * _L + lanes, zeros + ct])
                    rowbuf_v[k2, pl.ds(q * _L, _L)] = jnp.where(is_tail, tvec, vec)
                bv = plsc.load_gather(bslabs[s], [zeros, zeros + c])
                btv = plsc.load_gather(btail_v, [zeros, zeros + ct])
                bval = jnp.where(is_tail, btv, bv)
                if is_user:
                    slot = jnp.where(lanes == 0, bval,
                                     jnp.where(lanes == 1, ones_f, zeros_f))
                else:
                    slot = jnp.where(lanes == 1, bval,
                                     jnp.where(lanes == 0, ones_f, zeros_f))
                rowbuf_v[k2, pl.ds(N_FACTORS, _L)] = slot
                pos = plsc.load_gather(perm_v, [zeros + k2])[0]
                pltpu.make_async_copy(rowbuf_v.at[k2],
                                      out_hbm.at[pos], sem_out).start()
                return k2 + 1

            k = lax.while_loop(ext_cond, ext_body, k)
            bn = b + _RING
            @pl.when(bn <= b_hi)
            def _(s=s, bn=bn):
                fire(s, bn)
        return k

    nsteps = (b_hi - b_lo + _RING) // _RING
    lax.fori_loop(0, nsteps, ring_step, jnp.int32(0))

    pltpu.make_async_copy(out_hbm.at[pl.ds(0, _BPW)], rowbuf_v, sem_out).wait()


def _gather_a_body(ublk_hbm, ucol_hbm, pu_hbm, mblk_hbm, mcol_hbm, pm_hbm,
                   uft_hbm, mft_hbm, ub_hbm, mb_hbm,
                   urows_hbm, mrows_hbm,
                   blk_v, col_v, perm_v, rowbuf_v,
                   s0, s1, s2, s3, b0, b1, b2, b3,
                   tail_v, btail_v, sem, sem_out):
    slabs = (s0, s1, s2, s3)
    bslabs = (b0, b1, b2, b3)
    _extract_body(True, ublk_hbm, ucol_hbm, pu_hbm, uft_hbm, ub_hbm,
                  urows_hbm, blk_v, col_v, perm_v, rowbuf_v,
                  slabs, bslabs, tail_v, btail_v, sem, sem_out)
    _extract_body(False, mblk_hbm, mcol_hbm, pm_hbm, mft_hbm, mb_hbm,
                  mrows_hbm, blk_v, col_v, perm_v, rowbuf_v,
                  slabs, bslabs, tail_v, btail_v, sem, sem_out)


def _dot_body(urows_hbm, mrows_hbm, out_hbm,
              urows_v, mrows_v, out_v, pacc_v, sem_u, sem_m):
    wid = lax.axis_index("s") * _NC + lax.axis_index("c")
    base = wid * _BPW

    lanes = lax.iota(jnp.int32, _L)
    half = _BPW // 2

    def chunk(h, _):
        c0 = h * half
        cu = pltpu.make_async_copy(
            urows_hbm.at[pl.ds(base + c0, half)], urows_v, sem_u)
        cm = pltpu.make_async_copy(
            mrows_hbm.at[pl.ds(base + c0, half)], mrows_v, sem_m)
        cu.start(); cm.start(); cu.wait(); cm.wait()

        def block(b, _):
            r0 = b * _L
            for r in range(_L):
                acc = urows_v[r0 + r, pl.ds(0, _L)] * mrows_v[r0 + r, pl.ds(0, _L)]
                for j in range(1, EXT // _L):
                    acc = acc + (urows_v[r0 + r, pl.ds(j * _L, _L)]
                                 * mrows_v[r0 + r, pl.ds(j * _L, _L)])
                pacc_v[pl.ds(r * _L, _L)] = acc
            tot = plsc.load_gather(pacc_v, [lanes * _L])
            for l in range(1, _L):
                tot = tot + plsc.load_gather(pacc_v, [lanes * _L + l])
            out_v[pl.ds(c0 + r0, _L)] = tot
            return 0

        lax.fori_loop(0, half // _L, block, 0)
        return 0

    lax.fori_loop(0, 2, chunk, 0)
    pltpu.sync_copy(out_v, out_hbm.at[pl.ds(base, _BPW)])


@jax.jit
def kernel(user, movie, user_factors, movie_factors, user_biases, movie_biases):
    mesh = plsc.VectorSubcoreMesh(core_axis_name="c", subcore_axis_name="s")

    run_a = pl.kernel(
        _gather_a_body,
        out_type=(jax.ShapeDtypeStruct((BATCH, EXT), jnp.float32),
                  jax.ShapeDtypeStruct((BATCH, EXT), jnp.float32)),
        mesh=mesh,
        compiler_params=pltpu.CompilerParams(
            needs_layout_passes=False, use_tc_tiling_on_sc=True),
        scratch_types=[
            pltpu.VMEM((_BPW,), jnp.int32),
            pltpu.VMEM((_BPW,), jnp.int32),
            pltpu.VMEM((_BPW,), jnp.int32),
            pltpu.VMEM((_BPW, EXT), jnp.float32),
            pltpu.VMEM((N_FACTORS, 128), jnp.float32),
            pltpu.VMEM((N_FACTORS, 128), jnp.float32),
            pltpu.VMEM((N_FACTORS, 128), jnp.float32),
            pltpu.VMEM((N_FACTORS, 128), jnp.float32),
            pltpu.VMEM((1, 128), jnp.float32),
            pltpu.VMEM((1, 128), jnp.float32),
            pltpu.VMEM((1, 128), jnp.float32),
            pltpu.VMEM((1, 128), jnp.float32),
            pltpu.VMEM((N_FACTORS, _LASTW), jnp.float32),
            pltpu.VMEM((1, _LASTW), jnp.float32),
            pltpu.SemaphoreType.DMA,
            pltpu.SemaphoreType.DMA,
        ],
    )

    run_b = pl.kernel(
        _dot_body,
        out_type=jax.ShapeDtypeStruct((BATCH,), jnp.float32),
        mesh=mesh,
        compiler_params=pltpu.CompilerParams(
            needs_layout_passes=False, use_tc_tiling_on_sc=True),
        scratch_types=[
            pltpu.VMEM((_BPW // 2, EXT), jnp.float32),
            pltpu.VMEM((_BPW // 2, EXT), jnp.float32),
            pltpu.VMEM((_BPW,), jnp.float32),
            pltpu.VMEM((_L * _L,), jnp.float32),
            pltpu.SemaphoreType.DMA,
            pltpu.SemaphoreType.DMA,
        ],
    )

    pu = jnp.argsort(user).astype(jnp.int32)
    pm = jnp.argsort(movie).astype(jnp.int32)
    su = jnp.take(user, pu)
    sm = jnp.take(movie, pm)

    urows, mrows = run_a(su >> 7, su & 127, pu, sm >> 7, sm & 127, pm,
                         user_factors.T, movie_factors.T,
                         user_biases.T, movie_biases.T)
    return run_b(urows, mrows)


# speedup vs baseline: 1.1979x; 1.0580x over previous
import functools

import jax
import jax.numpy as jnp
from jax import lax
from jax.experimental import pallas as pl
from jax.experimental.pallas import tpu as pltpu
from jax.experimental.pallas import tpu_sc as plsc

N_FACTORS = 64
EXT = N_FACTORS + 16
BATCH = 16384
N_ROWS = 1000000

_info = plsc.get_sparse_core_info()
_NC, _NS, _L = _info.num_cores, _info.num_subcores, _info.num_lanes
_NW = _NC * _NS
_BPW = BATCH // _NW
_RING = 4
_NBLK = (N_ROWS + 127) // 128
_LASTW = N_ROWS - (_NBLK - 1) * 128


def _extract_body(is_user, blk_hbm, col_hbm, perm_hbm, ft_hbm, bias_hbm,
                  out_hbm, blk_v, col_v, perm_v, rowbuf_v,
                  slabs, bslabs, tail_v, btail_v, sem, sem_out):
    wid = lax.axis_index("s") * _NC + lax.axis_index("c")
    base = wid * _BPW
    lanes = lax.iota(jnp.int32, _L)
    zeros = lanes * 0
    ones_f = (zeros + 1).astype(jnp.float32)
    zeros_f = zeros.astype(jnp.float32)

    pltpu.sync_copy(blk_hbm.at[pl.ds(base, _BPW)], blk_v)
    pltpu.sync_copy(col_hbm.at[pl.ds(base, _BPW)], col_v)
    pltpu.sync_copy(perm_hbm.at[pl.ds(base, _BPW)], perm_v)

    b_lo = blk_v[pl.ds(0, _L)][0]
    b_hi = blk_v[pl.ds(_BPW - _L, _L)][_L - 1]

    def slab_copies(s, b):
        off = pl.multiple_of(b * 128, 128)
        full = pltpu.make_async_copy(
            ft_hbm.at[pl.ds(0, N_FACTORS), pl.ds(off, 128)], slabs[s], sem)
        bfull = pltpu.make_async_copy(
            bias_hbm.at[pl.ds(0, 1), pl.ds(off, 128)], bslabs[s], sem)
        tailoff = (_NBLK - 1) * 128
        tail = pltpu.make_async_copy(
            ft_hbm.at[pl.ds(0, N_FACTORS), pl.ds(tailoff, _LASTW)],
            tail_v, sem)
        btail = pltpu.make_async_copy(
            bias_hbm.at[pl.ds(0, 1), pl.ds(tailoff, _LASTW)], btail_v, sem)
        return full, bfull, tail, btail

    def fire(s, b):
        full, bfull, tail, btail = slab_copies(s, b)
        @pl.when(b < _NBLK - 1)
        def _():
            full.start(); bfull.start()
        @pl.when(b == _NBLK - 1)
        def _():
            tail.start(); btail.start()

    def drain(s, b):
        full, bfull, tail, btail = slab_copies(s, b)
        @pl.when((b <= b_hi) & (b < _NBLK - 1))
        def _():
            full.wait(); bfull.wait()
        @pl.when((b <= b_hi) & (b == _NBLK - 1))
        def _():
            tail.wait(); btail.wait()

    for s in range(_RING):
        @pl.when(b_lo + s <= b_hi)
        def _(s=s):
            fire(s, b_lo + s)

    def blk_at(k):
        v = plsc.load_gather(blk_v, [zeros + jnp.minimum(k, _BPW - 1)])[0]
        return jnp.where(k < _BPW, v, jnp.int32(-1))

    def ring_step(j, carry):
        k = carry
        for s in range(_RING):
            b = b_lo + j * _RING + s
            drain(s, b)

            def ext_cond(k2):
                return blk_at(k2) == b

            def finish_row(k2, bval):
                if is_user:
                    slot = jnp.where(lanes == 0, bval,
                                     jnp.where(lanes == 1, ones_f, zeros_f))
                else:
                    slot = jnp.where(lanes == 1, bval,
                                     jnp.where(lanes == 0, ones_f, zeros_f))
                rowbuf_v[k2, pl.ds(N_FACTORS, _L)] = slot
                pos = plsc.load_gather(perm_v, [zeros + k2])[0]
                pltpu.make_async_copy(rowbuf_v.at[k2],
                                      out_hbm.at[pos], sem_out).start()

            def ext_body(k2):
                c = plsc.load_gather(col_v, [zeros + k2])[0]
                for q in range(N_FACTORS // _L):
                    rowbuf_v[k2, pl.ds(q * _L, _L)] = plsc.load_gather(
                        slabs[s], [q * _L + lanes, zeros + c])
                finish_row(k2, plsc.load_gather(bslabs[s], [zeros, zeros + c]))
                return k2 + 1

            def ext_body_tail(k2):
                c = plsc.load_gather(col_v, [zeros + k2])[0]
                for q in range(N_FACTORS // _L):
                    rowbuf_v[k2, pl.ds(q * _L, _L)] = plsc.load_gather(
                        tail_v, [q * _L + lanes, zeros + c])
                finish_row(k2, plsc.load_gather(btail_v, [zeros, zeros + c]))
                return k2 + 1

            k = lax.cond(b == _NBLK - 1,
                         lambda kk: lax.while_loop(ext_cond, ext_body_tail, kk),
                         lambda kk: lax.while_loop(ext_cond, ext_body, kk),
                         k)
            bn = b + _RING
            @pl.when(bn <= b_hi)
            def _(s=s, bn=bn):
                fire(s, bn)
        return k

    nsteps = (b_hi - b_lo + _RING) // _RING
    lax.fori_loop(0, nsteps, ring_step, jnp.int32(0))

    pltpu.make_async_copy(out_hbm.at[pl.ds(0, _BPW)], rowbuf_v, sem_out).wait()


def _gather_a_body(ublk_hbm, ucol_hbm, pu_hbm, mblk_hbm, mcol_hbm, pm_hbm,
                   uft_hbm, mft_hbm, ub_hbm, mb_hbm,
                   urows_hbm, mrows_hbm,
                   blk_v, col_v, perm_v, rowbuf_v,
                   s0, s1, s2, s3, b0, b1, b2, b3,
                   tail_v, btail_v, sem, sem_out):
    slabs = (s0, s1, s2, s3)
    bslabs = (b0, b1, b2, b3)
    _extract_body(True, ublk_hbm, ucol_hbm, pu_hbm, uft_hbm, ub_hbm,
                  urows_hbm, blk_v, col_v, perm_v, rowbuf_v,
                  slabs, bslabs, tail_v, btail_v, sem, sem_out)
    _extract_body(False, mblk_hbm, mcol_hbm, pm_hbm, mft_hbm, mb_hbm,
                  mrows_hbm, blk_v, col_v, perm_v, rowbuf_v,
                  slabs, bslabs, tail_v, btail_v, sem, sem_out)


def _dot_body(urows_hbm, mrows_hbm, out_hbm,
              urows_v, mrows_v, out_v, pacc_v, sem_u, sem_m):
    wid = lax.axis_index("s") * _NC + lax.axis_index("c")
    base = wid * _BPW

    lanes = lax.iota(jnp.int32, _L)
    half = _BPW // 2

    def chunk(h, _):
        c0 = h * half
        cu = pltpu.make_async_copy(
            urows_hbm.at[pl.ds(base + c0, half)], urows_v, sem_u)
        cm = pltpu.make_async_copy(
            mrows_hbm.at[pl.ds(base + c0, half)], mrows_v, sem_m)
        cu.start(); cm.start(); cu.wait(); cm.wait()

        def block(b, _):
            r0 = b * _L
            for r in range(_L):
                acc = urows_v[r0 + r, pl.ds(0, _L)] * mrows_v[r0 + r, pl.ds(0, _L)]
                for j in range(1, EXT // _L):
                    acc = acc + (urows_v[r0 + r, pl.ds(j * _L, _L)]
                                 * mrows_v[r0 + r, pl.ds(j * _L, _L)])
                pacc_v[pl.ds(r * _L, _L)] = acc
            tot = plsc.load_gather(pacc_v, [lanes * _L])
            for l in range(1, _L):
                tot = tot + plsc.load_gather(pacc_v, [lanes * _L + l])
            out_v[pl.ds(c0 + r0, _L)] = tot
            return 0

        lax.fori_loop(0, half // _L, block, 0)
        return 0

    lax.fori_loop(0, 2, chunk, 0)
    pltpu.sync_copy(out_v, out_hbm.at[pl.ds(base, _BPW)])


@jax.jit
def kernel(user, movie, user_factors, movie_factors, user_biases, movie_biases):
    mesh = plsc.VectorSubcoreMesh(core_axis_name="c", subcore_axis_name="s")

    run_a = pl.kernel(
        _gather_a_body,
        out_type=(jax.ShapeDtypeStruct((BATCH, EXT), jnp.float32),
                  jax.ShapeDtypeStruct((BATCH, EXT), jnp.float32)),
        mesh=mesh,
        compiler_params=pltpu.CompilerParams(
            needs_layout_passes=False, use_tc_tiling_on_sc=True),
        scratch_types=[
            pltpu.VMEM((_BPW,), jnp.int32),
            pltpu.VMEM((_BPW,), jnp.int32),
            pltpu.VMEM((_BPW,), jnp.int32),
            pltpu.VMEM((_BPW, EXT), jnp.float32),
            pltpu.VMEM((N_FACTORS, 128), jnp.float32),
            pltpu.VMEM((N_FACTORS, 128), jnp.float32),
            pltpu.VMEM((N_FACTORS, 128), jnp.float32),
            pltpu.VMEM((N_FACTORS, 128), jnp.float32),
            pltpu.VMEM((1, 128), jnp.float32),
            pltpu.VMEM((1, 128), jnp.float32),
            pltpu.VMEM((1, 128), jnp.float32),
            pltpu.VMEM((1, 128), jnp.float32),
            pltpu.VMEM((N_FACTORS, _LASTW), jnp.float32),
            pltpu.VMEM((1, _LASTW), jnp.float32),
            pltpu.SemaphoreType.DMA,
            pltpu.SemaphoreType.DMA,
        ],
    )

    run_b = pl.kernel(
        _dot_body,
        out_type=jax.ShapeDtypeStruct((BATCH,), jnp.float32),
        mesh=mesh,
        compiler_params=pltpu.CompilerParams(
            needs_layout_passes=False, use_tc_tiling_on_sc=True),
        scratch_types=[
            pltpu.VMEM((_BPW // 2, EXT), jnp.float32),
            pltpu.VMEM((_BPW // 2, EXT), jnp.float32),
            pltpu.VMEM((_BPW,), jnp.float32),
            pltpu.VMEM((_L * _L,), jnp.float32),
            pltpu.SemaphoreType.DMA,
            pltpu.SemaphoreType.DMA,
        ],
    )

    pu = jnp.argsort(user).astype(jnp.int32)
    pm = jnp.argsort(movie).astype(jnp.int32)
    su = jnp.take(user, pu)
    sm = jnp.take(movie, pm)

    urows, mrows = run_a(su >> 7, su & 127, pu, sm >> 7, sm & 127, pm,
                         user_factors.T, movie_factors.T,
                         user_biases.T, movie_biases.T)
    return run_b(urows, mrows)


# speedup vs baseline: 1.2851x; 1.0728x over previous
import functools

import jax
import jax.numpy as jnp
from jax import lax
from jax.experimental import pallas as pl
from jax.experimental.pallas import tpu as pltpu
from jax.experimental.pallas import tpu_sc as plsc

N_FACTORS = 64
EXT = N_FACTORS + 16
BATCH = 16384
N_ROWS = 1000000

_info = plsc.get_sparse_core_info()
_NC, _NS, _L = _info.num_cores, _info.num_subcores, _info.num_lanes
_NW = _NC * _NS
_BPW = BATCH // _NW
_RING = 5
_NBLK = (N_ROWS + 127) // 128
_LASTW = N_ROWS - (_NBLK - 1) * 128


def _extract_body(is_user, blk_hbm, col_hbm, perm_hbm, ft_hbm, bias_hbm,
                  out_hbm, blk_v, col_v, perm_v, rowbuf_v,
                  slabs, bslabs, tail_v, btail_v, sem, sem_out):
    wid = lax.axis_index("s") * _NC + lax.axis_index("c")
    base = wid * _BPW
    lanes = lax.iota(jnp.int32, _L)
    zeros = lanes * 0
    ones_f = (zeros + 1).astype(jnp.float32)
    zeros_f = zeros.astype(jnp.float32)

    pltpu.sync_copy(blk_hbm.at[pl.ds(base, _BPW)], blk_v)
    pltpu.sync_copy(col_hbm.at[pl.ds(base, _BPW)], col_v)
    pltpu.sync_copy(perm_hbm.at[pl.ds(base, _BPW)], perm_v)

    b_lo = blk_v[pl.ds(0, _L)][0]
    b_hi = blk_v[pl.ds(_BPW - _L, _L)][_L - 1]

    def slab_copies(s, b):
        off = pl.multiple_of(b * 128, 128)
        full = pltpu.make_async_copy(
            ft_hbm.at[pl.ds(0, N_FACTORS), pl.ds(off, 128)], slabs[s], sem)
        bfull = pltpu.make_async_copy(
            bias_hbm.at[pl.ds(0, 1), pl.ds(off, 128)], bslabs[s], sem)
        tailoff = (_NBLK - 1) * 128
        tail = pltpu.make_async_copy(
            ft_hbm.at[pl.ds(0, N_FACTORS), pl.ds(tailoff, _LASTW)],
            tail_v, sem)
        btail = pltpu.make_async_copy(
            bias_hbm.at[pl.ds(0, 1), pl.ds(tailoff, _LASTW)], btail_v, sem)
        return full, bfull, tail, btail

    def fire(s, b):
        full, bfull, tail, btail = slab_copies(s, b)
        @pl.when(b < _NBLK - 1)
        def _():
            full.start(); bfull.start()
        @pl.when(b == _NBLK - 1)
        def _():
            tail.start(); btail.start()

    def drain(s, b):
        full, bfull, tail, btail = slab_copies(s, b)
        @pl.when((b <= b_hi) & (b < _NBLK - 1))
        def _():
            full.wait(); bfull.wait()
        @pl.when((b <= b_hi) & (b == _NBLK - 1))
        def _():
            tail.wait(); btail.wait()

    for s in range(_RING):
        @pl.when(b_lo + s <= b_hi)
        def _(s=s):
            fire(s, b_lo + s)

    def blk_at(k):
        v = plsc.load_gather(blk_v, [zeros + jnp.minimum(k, _BPW - 1)])[0]
        return jnp.where(k < _BPW, v, jnp.int32(-1))

    def ring_step(j, carry):
        k = carry
        for s in range(_RING):
            b = b_lo + j * _RING + s
            drain(s, b)

            def ext_cond(k2):
                return blk_at(k2) == b

            def finish_row(k2, bval):
                if is_user:
                    slot = jnp.where(lanes == 0, bval,
                                     jnp.where(lanes == 1, ones_f, zeros_f))
                else:
                    slot = jnp.where(lanes == 1, bval,
                                     jnp.where(lanes == 0, ones_f, zeros_f))
                rowbuf_v[k2, pl.ds(N_FACTORS, _L)] = slot
                pos = plsc.load_gather(perm_v, [zeros + k2])[0]
                pltpu.make_async_copy(rowbuf_v.at[k2],
                                      out_hbm.at[pos], sem_out).start()

            def ext_body(k2):
                c = plsc.load_gather(col_v, [zeros + k2])[0]
                for q in range(N_FACTORS // _L):
                    rowbuf_v[k2, pl.ds(q * _L, _L)] = plsc.load_gather(
                        slabs[s], [q * _L + lanes, zeros + c])
                finish_row(k2, plsc.load_gather(bslabs[s], [zeros, zeros + c]))
                return k2 + 1

            def ext_body_tail(k2):
                c = plsc.load_gather(col_v, [zeros + k2])[0]
                for q in range(N_FACTORS // _L):
                    rowbuf_v[k2, pl.ds(q * _L, _L)] = plsc.load_gather(
                        tail_v, [q * _L + lanes, zeros + c])
                finish_row(k2, plsc.load_gather(btail_v, [zeros, zeros + c]))
                return k2 + 1

            k = lax.cond(b == _NBLK - 1,
                         lambda kk: lax.while_loop(ext_cond, ext_body_tail, kk),
                         lambda kk: lax.while_loop(ext_cond, ext_body, kk),
                         k)
            bn = b + _RING
            @pl.when(bn <= b_hi)
            def _(s=s, bn=bn):
                fire(s, bn)
        return k

    nsteps = (b_hi - b_lo + _RING) // _RING
    lax.fori_loop(0, nsteps, ring_step, jnp.int32(0))

    pltpu.make_async_copy(out_hbm.at[pl.ds(0, _BPW)], rowbuf_v, sem_out).wait()


def _gather_a_body(ublk_hbm, ucol_hbm, pu_hbm, mblk_hbm, mcol_hbm, pm_hbm,
                   uft_hbm, mft_hbm, ub_hbm, mb_hbm,
                   urows_hbm, mrows_hbm,
                   blk_v, col_v, perm_v, rowbuf_v,
                   s0, s1, s2, s3, s4, b0, b1, b2, b3, b4,
                   tail_v, btail_v, sem, sem_out):
    slabs = (s0, s1, s2, s3, s4)
    bslabs = (b0, b1, b2, b3, b4)
    _extract_body(True, ublk_hbm, ucol_hbm, pu_hbm, uft_hbm, ub_hbm,
                  urows_hbm, blk_v, col_v, perm_v, rowbuf_v,
                  slabs, bslabs, tail_v, btail_v, sem, sem_out)
    _extract_body(False, mblk_hbm, mcol_hbm, pm_hbm, mft_hbm, mb_hbm,
                  mrows_hbm, blk_v, col_v, perm_v, rowbuf_v,
                  slabs, bslabs, tail_v, btail_v, sem, sem_out)


def _dot_body(urows_hbm, mrows_hbm, out_hbm,
              urows_v, mrows_v, out_v, pacc_v, sem_u, sem_m):
    wid = lax.axis_index("s") * _NC + lax.axis_index("c")
    base = wid * _BPW

    lanes = lax.iota(jnp.int32, _L)
    half = _BPW // 2

    def chunk(h, _):
        c0 = h * half
        cu = pltpu.make_async_copy(
            urows_hbm.at[pl.ds(base + c0, half)], urows_v, sem_u)
        cm = pltpu.make_async_copy(
            mrows_hbm.at[pl.ds(base + c0, half)], mrows_v, sem_m)
        cu.start(); cm.start(); cu.wait(); cm.wait()

        def block(b, _):
            r0 = b * _L
            for r in range(_L):
                acc = urows_v[r0 + r, pl.ds(0, _L)] * mrows_v[r0 + r, pl.ds(0, _L)]
                for j in range(1, EXT // _L):
                    acc = acc + (urows_v[r0 + r, pl.ds(j * _L, _L)]
                                 * mrows_v[r0 + r, pl.ds(j * _L, _L)])
                pacc_v[pl.ds(r * _L, _L)] = acc
            tot = plsc.load_gather(pacc_v, [lanes * _L])
            for l in range(1, _L):
                tot = tot + plsc.load_gather(pacc_v, [lanes * _L + l])
            out_v[pl.ds(c0 + r0, _L)] = tot
            return 0

        lax.fori_loop(0, half // _L, block, 0)
        return 0

    lax.fori_loop(0, 2, chunk, 0)
    pltpu.sync_copy(out_v, out_hbm.at[pl.ds(base, _BPW)])


@jax.jit
def kernel(user, movie, user_factors, movie_factors, user_biases, movie_biases):
    mesh = plsc.VectorSubcoreMesh(core_axis_name="c", subcore_axis_name="s")

    run_a = pl.kernel(
        _gather_a_body,
        out_type=(jax.ShapeDtypeStruct((BATCH, EXT), jnp.float32),
                  jax.ShapeDtypeStruct((BATCH, EXT), jnp.float32)),
        mesh=mesh,
        compiler_params=pltpu.CompilerParams(
            needs_layout_passes=False, use_tc_tiling_on_sc=True),
        scratch_types=[
            pltpu.VMEM((_BPW,), jnp.int32),
            pltpu.VMEM((_BPW,), jnp.int32),
            pltpu.VMEM((_BPW,), jnp.int32),
            pltpu.VMEM((_BPW, EXT), jnp.float32),
            pltpu.VMEM((N_FACTORS, 128), jnp.float32),
            pltpu.VMEM((N_FACTORS, 128), jnp.float32),
            pltpu.VMEM((N_FACTORS, 128), jnp.float32),
            pltpu.VMEM((N_FACTORS, 128), jnp.float32),
            pltpu.VMEM((N_FACTORS, 128), jnp.float32),
            pltpu.VMEM((1, 128), jnp.float32),
            pltpu.VMEM((1, 128), jnp.float32),
            pltpu.VMEM((1, 128), jnp.float32),
            pltpu.VMEM((1, 128), jnp.float32),
            pltpu.VMEM((1, 128), jnp.float32),
            pltpu.VMEM((N_FACTORS, _LASTW), jnp.float32),
            pltpu.VMEM((1, _LASTW), jnp.float32),
            pltpu.SemaphoreType.DMA,
            pltpu.SemaphoreType.DMA,
        ],
    )

    run_b = pl.kernel(
        _dot_body,
        out_type=jax.ShapeDtypeStruct((BATCH,), jnp.float32),
        mesh=mesh,
        compiler_params=pltpu.CompilerParams(
            needs_layout_passes=False, use_tc_tiling_on_sc=True),
        scratch_types=[
            pltpu.VMEM((_BPW // 2, EXT), jnp.float32),
            pltpu.VMEM((_BPW // 2, EXT), jnp.float32),
            pltpu.VMEM((_BPW,), jnp.float32),
            pltpu.VMEM((_L * _L,), jnp.float32),
            pltpu.SemaphoreType.DMA,
            pltpu.SemaphoreType.DMA,
        ],
    )

    pu = jnp.argsort(user).astype(jnp.int32)
    pm = jnp.argsort(movie).astype(jnp.int32)
    su = jnp.take(user, pu)
    sm = jnp.take(movie, pm)

    urows, mrows = run_a(su >> 7, su & 127, pu, sm >> 7, sm & 127, pm,
                         user_factors.T, movie_factors.T,
                         user_biases.T, movie_biases.T)
    return run_b(urows, mrows)


# speedup vs baseline: 1.3113x; 1.0204x over previous
import functools

import jax
import jax.numpy as jnp
from jax import lax
from jax.experimental import pallas as pl
from jax.experimental.pallas import tpu as pltpu
from jax.experimental.pallas import tpu_sc as plsc

N_FACTORS = 64
EXT = N_FACTORS + 16
BATCH = 16384
N_ROWS = 1000000

_info = plsc.get_sparse_core_info()
_NC, _NS, _L = _info.num_cores, _info.num_subcores, _info.num_lanes
_NW = _NC * _NS
_BPW = BATCH // _NW
_RING = 6
_NBLK = (N_ROWS + 127) // 128
_LASTW = N_ROWS - (_NBLK - 1) * 128


def _extract_body(is_user, blk_hbm, col_hbm, perm_hbm, ft_hbm, bias_hbm,
                  out_hbm, blk_v, col_v, perm_v, rowbuf_v,
                  slabs, bslabs, tail_v, btail_v, sem, sem_out):
    wid = lax.axis_index("s") * _NC + lax.axis_index("c")
    base = wid * _BPW
    lanes = lax.iota(jnp.int32, _L)
    zeros = lanes * 0
    ones_f = (zeros + 1).astype(jnp.float32)
    zeros_f = zeros.astype(jnp.float32)

    pltpu.sync_copy(blk_hbm.at[pl.ds(base, _BPW)], blk_v)
    pltpu.sync_copy(col_hbm.at[pl.ds(base, _BPW)], col_v)
    pltpu.sync_copy(perm_hbm.at[pl.ds(base, _BPW)], perm_v)

    b_lo = blk_v[pl.ds(0, _L)][0]
    b_hi = blk_v[pl.ds(_BPW - _L, _L)][_L - 1]

    def slab_copies(s, b):
        off = pl.multiple_of(b * 128, 128)
        full = pltpu.make_async_copy(
            ft_hbm.at[pl.ds(0, N_FACTORS), pl.ds(off, 128)], slabs[s], sem)
        bfull = pltpu.make_async_copy(
            bias_hbm.at[pl.ds(0, 1), pl.ds(off, 128)], bslabs[s], sem)
        tailoff = (_NBLK - 1) * 128
        tail = pltpu.make_async_copy(
            ft_hbm.at[pl.ds(0, N_FACTORS), pl.ds(tailoff, _LASTW)],
            tail_v, sem)
        btail = pltpu.make_async_copy(
            bias_hbm.at[pl.ds(0, 1), pl.ds(tailoff, _LASTW)], btail_v, sem)
        return full, bfull, tail, btail

    def fire(s, b):
        full, bfull, tail, btail = slab_copies(s, b)
        @pl.when(b < _NBLK - 1)
        def _():
            full.start(); bfull.start()
        @pl.when(b == _NBLK - 1)
        def _():
            tail.start(); btail.start()

    def drain(s, b):
        full, bfull, tail, btail = slab_copies(s, b)
        @pl.when((b <= b_hi) & (b < _NBLK - 1))
        def _():
            full.wait(); bfull.wait()
        @pl.when((b <= b_hi) & (b == _NBLK - 1))
        def _():
            tail.wait(); btail.wait()

    for s in range(_RING):
        @pl.when(b_lo + s <= b_hi)
        def _(s=s):
            fire(s, b_lo + s)

    def blk_at(k):
        v = plsc.load_gather(blk_v, [zeros + jnp.minimum(k, _BPW - 1)])[0]
        return jnp.where(k < _BPW, v, jnp.int32(-1))

    def ring_step(j, carry):
        k = carry
        for s in range(_RING):
            b = b_lo + j * _RING + s
            drain(s, b)

            def ext_cond(k2):
                return blk_at(k2) == b

            def finish_row(k2, bval):
                if is_user:
                    slot = jnp.where(lanes == 0, bval,
                                     jnp.where(lanes == 1, ones_f, zeros_f))
                else:
                    slot = jnp.where(lanes == 1, bval,
                                     jnp.where(lanes == 0, ones_f, zeros_f))
                rowbuf_v[k2, pl.ds(N_FACTORS, _L)] = slot
                pos = plsc.load_gather(perm_v, [zeros + k2])[0]
                pltpu.make_async_copy(rowbuf_v.at[k2],
                                      out_hbm.at[pos], sem_out).start()

            def ext_body(k2):
                c = plsc.load_gather(col_v, [zeros + k2])[0]
                for q in range(N_FACTORS // _L):
                    rowbuf_v[k2, pl.ds(q * _L, _L)] = plsc.load_gather(
                        slabs[s], [q * _L + lanes, zeros + c])
                finish_row(k2, plsc.load_gather(bslabs[s], [zeros, zeros + c]))
                return k2 + 1

            def ext_body_tail(k2):
                c = plsc.load_gather(col_v, [zeros + k2])[0]
                for q in range(N_FACTORS // _L):
                    rowbuf_v[k2, pl.ds(q * _L, _L)] = plsc.load_gather(
                        tail_v, [q * _L + lanes, zeros + c])
                finish_row(k2, plsc.load_gather(btail_v, [zeros, zeros + c]))
                return k2 + 1

            k = lax.cond(b == _NBLK - 1,
                         lambda kk: lax.while_loop(ext_cond, ext_body_tail, kk),
                         lambda kk: lax.while_loop(ext_cond, ext_body, kk),
                         k)
            bn = b + _RING
            @pl.when(bn <= b_hi)
            def _(s=s, bn=bn):
                fire(s, bn)
        return k

    nsteps = (b_hi - b_lo + _RING) // _RING
    lax.fori_loop(0, nsteps, ring_step, jnp.int32(0))

    pltpu.make_async_copy(out_hbm.at[pl.ds(0, _BPW)], rowbuf_v, sem_out).wait()


def _gather_a_body(ublk_hbm, ucol_hbm, pu_hbm, mblk_hbm, mcol_hbm, pm_hbm,
                   uft_hbm, mft_hbm, ub_hbm, mb_hbm,
                   urows_hbm, mrows_hbm,
                   blk_v, col_v, perm_v, rowbuf_v,
                   s0, s1, s2, s3, s4, s5, b0, b1, b2, b3, b4, b5,
                   tail_v, btail_v, sem, sem_out):
    slabs = (s0, s1, s2, s3, s4, s5)
    bslabs = (b0, b1, b2, b3, b4, b5)
    _extract_body(True, ublk_hbm, ucol_hbm, pu_hbm, uft_hbm, ub_hbm,
                  urows_hbm, blk_v, col_v, perm_v, rowbuf_v,
                  slabs, bslabs, tail_v, btail_v, sem, sem_out)
    _extract_body(False, mblk_hbm, mcol_hbm, pm_hbm, mft_hbm, mb_hbm,
                  mrows_hbm, blk_v, col_v, perm_v, rowbuf_v,
                  slabs, bslabs, tail_v, btail_v, sem, sem_out)


def _dot_body(urows_hbm, mrows_hbm, out_hbm,
              urows_v, mrows_v, out_v, pacc_v, sem_u, sem_m):
    wid = lax.axis_index("s") * _NC + lax.axis_index("c")
    base = wid * _BPW

    lanes = lax.iota(jnp.int32, _L)
    half = _BPW // 2

    def chunk(h, _):
        c0 = h * half
        cu = pltpu.make_async_copy(
            urows_hbm.at[pl.ds(base + c0, half)], urows_v, sem_u)
        cm = pltpu.make_async_copy(
            mrows_hbm.at[pl.ds(base + c0, half)], mrows_v, sem_m)
        cu.start(); cm.start(); cu.wait(); cm.wait()

        def block(b, _):
            r0 = b * _L
            for r in range(_L):
                acc = urows_v[r0 + r, pl.ds(0, _L)] * mrows_v[r0 + r, pl.ds(0, _L)]
                for j in range(1, EXT // _L):
                    acc = acc + (urows_v[r0 + r, pl.ds(j * _L, _L)]
                                 * mrows_v[r0 + r, pl.ds(j * _L, _L)])
                pacc_v[pl.ds(r * _L, _L)] = acc
            tot = plsc.load_gather(pacc_v, [lanes * _L])
            for l in range(1, _L):
                tot = tot + plsc.load_gather(pacc_v, [lanes * _L + l])
            out_v[pl.ds(c0 + r0, _L)] = tot
            return 0

        lax.fori_loop(0, half // _L, block, 0)
        return 0

    lax.fori_loop(0, 2, chunk, 0)
    pltpu.sync_copy(out_v, out_hbm.at[pl.ds(base, _BPW)])


@jax.jit
def kernel(user, movie, user_factors, movie_factors, user_biases, movie_biases):
    mesh = plsc.VectorSubcoreMesh(core_axis_name="c", subcore_axis_name="s")

    run_a = pl.kernel(
        _gather_a_body,
        out_type=(jax.ShapeDtypeStruct((BATCH, EXT), jnp.float32),
                  jax.ShapeDtypeStruct((BATCH, EXT), jnp.float32)),
        mesh=mesh,
        compiler_params=pltpu.CompilerParams(
            needs_layout_passes=False, use_tc_tiling_on_sc=True),
        scratch_types=[
            pltpu.VMEM((_BPW,), jnp.int32),
            pltpu.VMEM((_BPW,), jnp.int32),
            pltpu.VMEM((_BPW,), jnp.int32),
            pltpu.VMEM((_BPW, EXT), jnp.float32),
            pltpu.VMEM((N_FACTORS, 128), jnp.float32),
            pltpu.VMEM((N_FACTORS, 128), jnp.float32),
            pltpu.VMEM((N_FACTORS, 128), jnp.float32),
            pltpu.VMEM((N_FACTORS, 128), jnp.float32),
            pltpu.VMEM((N_FACTORS, 128), jnp.float32),
            pltpu.VMEM((N_FACTORS, 128), jnp.float32),
            pltpu.VMEM((1, 128), jnp.float32),
            pltpu.VMEM((1, 128), jnp.float32),
            pltpu.VMEM((1, 128), jnp.float32),
            pltpu.VMEM((1, 128), jnp.float32),
            pltpu.VMEM((1, 128), jnp.float32),
            pltpu.VMEM((1, 128), jnp.float32),
            pltpu.VMEM((N_FACTORS, _LASTW), jnp.float32),
            pltpu.VMEM((1, _LASTW), jnp.float32),
            pltpu.SemaphoreType.DMA,
            pltpu.SemaphoreType.DMA,
        ],
    )

    run_b = pl.kernel(
        _dot_body,
        out_type=jax.ShapeDtypeStruct((BATCH,), jnp.float32),
        mesh=mesh,
        compiler_params=pltpu.CompilerParams(
            needs_layout_passes=False, use_tc_tiling_on_sc=True),
        scratch_types=[
            pltpu.VMEM((_BPW // 2, EXT), jnp.float32),
            pltpu.VMEM((_BPW // 2, EXT), jnp.float32),
            pltpu.VMEM((_BPW,), jnp.float32),
            pltpu.VMEM((_L * _L,), jnp.float32),
            pltpu.SemaphoreType.DMA,
            pltpu.SemaphoreType.DMA,
        ],
    )

    pu = jnp.argsort(user).astype(jnp.int32)
    pm = jnp.argsort(movie).astype(jnp.int32)
    su = jnp.take(user, pu)
    sm = jnp.take(movie, pm)

    urows, mrows = run_a(su >> 7, su & 127, pu, sm >> 7, sm & 127, pm,
                         user_factors.T, movie_factors.T,
                         user_biases.T, movie_biases.T)
    return run_b(urows, mrows)


# speedup vs baseline: 1.3983x; 1.0664x over previous
import functools

import jax
import jax.numpy as jnp
from jax import lax
from jax.experimental import pallas as pl
from jax.experimental.pallas import tpu as pltpu
from jax.experimental.pallas import tpu_sc as plsc

N_FACTORS = 64
EXT = N_FACTORS + 16
BATCH = 16384
N_ROWS = 1000000

_info = plsc.get_sparse_core_info()
_NC, _NS, _L = _info.num_cores, _info.num_subcores, _info.num_lanes
_NW = _NC * _NS
_BPW = BATCH // _NW
_RING = 6
_NBLK = (N_ROWS + 127) // 128
_LASTW = N_ROWS - (_NBLK - 1) * 128


def _extract_body(is_user, blk_hbm, col_hbm, perm_hbm, ft_hbm, bias_hbm,
                  out_hbm, blk_v, col_v, perm_v, rowbuf_v,
                  slabs, bslabs, tail_v, btail_v, sem, sem_out):
    wid = lax.axis_index("s") * _NC + lax.axis_index("c")
    base = wid * _BPW
    lanes = lax.iota(jnp.int32, _L)
    zeros = lanes * 0
    ones_f = (zeros + 1).astype(jnp.float32)
    zeros_f = zeros.astype(jnp.float32)

    pltpu.sync_copy(blk_hbm.at[pl.ds(base, _BPW)], blk_v)
    pltpu.sync_copy(col_hbm.at[pl.ds(base, _BPW)], col_v)
    pltpu.sync_copy(perm_hbm.at[pl.ds(base, _BPW)], perm_v)

    b_lo = blk_v[pl.ds(0, _L)][0]
    b_hi = blk_v[pl.ds(_BPW - _L, _L)][_L - 1]

    def slab_copies(s, b):
        off = pl.multiple_of(b * 128, 128)
        full = pltpu.make_async_copy(
            ft_hbm.at[pl.ds(0, N_FACTORS), pl.ds(off, 128)], slabs[s], sem)
        bfull = pltpu.make_async_copy(
            bias_hbm.at[pl.ds(0, 1), pl.ds(off, 128)], bslabs[s], sem)
        tailoff = (_NBLK - 1) * 128
        tail = pltpu.make_async_copy(
            ft_hbm.at[pl.ds(0, N_FACTORS), pl.ds(tailoff, _LASTW)],
            tail_v, sem)
        btail = pltpu.make_async_copy(
            bias_hbm.at[pl.ds(0, 1), pl.ds(tailoff, _LASTW)], btail_v, sem)
        return full, bfull, tail, btail

    def fire(s, b):
        full, bfull, tail, btail = slab_copies(s, b)
        @pl.when(b < _NBLK - 1)
        def _():
            full.start(); bfull.start()
        @pl.when(b == _NBLK - 1)
        def _():
            tail.start(); btail.start()

    def drain(s, b):
        full, bfull, tail, btail = slab_copies(s, b)
        @pl.when((b <= b_hi) & (b < _NBLK - 1))
        def _():
            full.wait(); bfull.wait()
        @pl.when((b <= b_hi) & (b == _NBLK - 1))
        def _():
            tail.wait(); btail.wait()

    for s in range(_RING):
        @pl.when(b_lo + s <= b_hi)
        def _(s=s):
            fire(s, b_lo + s)

    def blk_at(k):
        v = plsc.load_gather(blk_v, [zeros + jnp.minimum(k, _BPW - 1)])[0]
        return jnp.where(k < _BPW, v, jnp.int32(-1))

    def ring_step(j, carry):
        k = carry
        for s in range(_RING):
            b = b_lo + j * _RING + s
            drain(s, b)

            def ext_cond(k2):
                return blk_at(k2) == b

            def finish_row(k2, bval):
                if is_user:
                    slot = jnp.where(lanes == 0, bval,
                                     jnp.where(lanes == 1, ones_f, zeros_f))
                else:
                    slot = jnp.where(lanes == 1, bval,
                                     jnp.where(lanes == 0, ones_f, zeros_f))
                rowbuf_v[k2, pl.ds(N_FACTORS, _L)] = slot
                pos = plsc.load_gather(perm_v, [zeros + k2])[0]
                pltpu.make_async_copy(rowbuf_v.at[k2],
                                      out_hbm.at[pos], sem_out).start()

            def ext_body(k2):
                c = plsc.load_gather(col_v, [zeros + k2])[0]
                for q in range(N_FACTORS // _L):
                    rowbuf_v[k2, pl.ds(q * _L, _L)] = plsc.load_gather(
                        slabs[s], [q * _L + lanes, zeros + c])
                finish_row(k2, plsc.load_gather(bslabs[s], [zeros, zeros + c]))
                return k2 + 1

            def ext_body_tail(k2):
                c = plsc.load_gather(col_v, [zeros + k2])[0]
                for q in range(N_FACTORS // _L):
                    rowbuf_v[k2, pl.ds(q * _L, _L)] = plsc.load_gather(
                        tail_v, [q * _L + lanes, zeros + c])
                finish_row(k2, plsc.load_gather(btail_v, [zeros, zeros + c]))
                return k2 + 1

            k = lax.cond(b == _NBLK - 1,
                         lambda kk: lax.while_loop(ext_cond, ext_body_tail, kk),
                         lambda kk: lax.while_loop(ext_cond, ext_body, kk),
                         k)
            bn = b + _RING
            @pl.when(bn <= b_hi)
            def _(s=s, bn=bn):
                fire(s, bn)
        return k

    nsteps = (b_hi - b_lo + _RING) // _RING
    lax.fori_loop(0, nsteps, ring_step, jnp.int32(0))

    pltpu.make_async_copy(out_hbm.at[pl.ds(0, _BPW)], rowbuf_v, sem_out).wait()


def _gather_a_body(ublk_hbm, ucol_hbm, pu_hbm, mblk_hbm, mcol_hbm, pm_hbm,
                   uft_hbm, mft_hbm, ub_hbm, mb_hbm,
                   urows_hbm, mrows_hbm,
                   blk_v, col_v, perm_v, rowbuf_v,
                   s0, s1, s2, s3, s4, s5, b0, b1, b2, b3, b4, b5,
                   tail_v, btail_v, sem, sem_out):
    slabs = (s0, s1, s2, s3, s4, s5)
    bslabs = (b0, b1, b2, b3, b4, b5)
    _extract_body(True, ublk_hbm, ucol_hbm, pu_hbm, uft_hbm, ub_hbm,
                  urows_hbm, blk_v, col_v, perm_v, rowbuf_v,
                  slabs, bslabs, tail_v, btail_v, sem, sem_out)
    _extract_body(False, mblk_hbm, mcol_hbm, pm_hbm, mft_hbm, mb_hbm,
                  mrows_hbm, blk_v, col_v, perm_v, rowbuf_v,
                  slabs, bslabs, tail_v, btail_v, sem, sem_out)


def _dot_body(urows_hbm, mrows_hbm, out_hbm,
              urows_v, mrows_v, out_v, pacc_v, sem_u, sem_m):
    wid = lax.axis_index("s") * _NC + lax.axis_index("c")
    base = wid * _BPW

    lanes = lax.iota(jnp.int32, _L)
    half = _BPW // 2

    def chunk(h, _):
        c0 = h * half
        cu = pltpu.make_async_copy(
            urows_hbm.at[pl.ds(base + c0, half)], urows_v, sem_u)
        cm = pltpu.make_async_copy(
            mrows_hbm.at[pl.ds(base + c0, half)], mrows_v, sem_m)
        cu.start(); cm.start(); cu.wait(); cm.wait()

        def block(b, _):
            r0 = b * _L
            for r in range(_L):
                acc = urows_v[r0 + r, pl.ds(0, _L)] * mrows_v[r0 + r, pl.ds(0, _L)]
                for j in range(1, EXT // _L):
                    acc = acc + (urows_v[r0 + r, pl.ds(j * _L, _L)]
                                 * mrows_v[r0 + r, pl.ds(j * _L, _L)])
                pacc_v[pl.ds(r * _L, _L)] = acc
            tot = plsc.load_gather(pacc_v, [lanes * _L])
            for l in range(1, _L):
                tot = tot + plsc.load_gather(pacc_v, [lanes * _L + l])
            out_v[pl.ds(c0 + r0, _L)] = tot
            return 0

        lax.fori_loop(0, half // _L, block, 0)
        return 0

    lax.fori_loop(0, 2, chunk, 0)
    pltpu.sync_copy(out_v, out_hbm.at[pl.ds(base, _BPW)])


@jax.jit
def kernel(user, movie, user_factors, movie_factors, user_biases, movie_biases):
    mesh = plsc.VectorSubcoreMesh(core_axis_name="c", subcore_axis_name="s")

    run_a = pl.kernel(
        _gather_a_body,
        out_type=(jax.ShapeDtypeStruct((BATCH, EXT), jnp.float32),
                  jax.ShapeDtypeStruct((BATCH, EXT), jnp.float32)),
        mesh=mesh,
        compiler_params=pltpu.CompilerParams(
            needs_layout_passes=False, use_tc_tiling_on_sc=True),
        scratch_types=[
            pltpu.VMEM((_BPW,), jnp.int32),
            pltpu.VMEM((_BPW,), jnp.int32),
            pltpu.VMEM((_BPW,), jnp.int32),
            pltpu.VMEM((_BPW, EXT), jnp.float32),
            pltpu.VMEM((N_FACTORS, 128), jnp.float32),
            pltpu.VMEM((N_FACTORS, 128), jnp.float32),
            pltpu.VMEM((N_FACTORS, 128), jnp.float32),
            pltpu.VMEM((N_FACTORS, 128), jnp.float32),
            pltpu.VMEM((N_FACTORS, 128), jnp.float32),
            pltpu.VMEM((N_FACTORS, 128), jnp.float32),
            pltpu.VMEM((1, 128), jnp.float32),
            pltpu.VMEM((1, 128), jnp.float32),
            pltpu.VMEM((1, 128), jnp.float32),
            pltpu.VMEM((1, 128), jnp.float32),
            pltpu.VMEM((1, 128), jnp.float32),
            pltpu.VMEM((1, 128), jnp.float32),
            pltpu.VMEM((N_FACTORS, _LASTW), jnp.float32),
            pltpu.VMEM((1, _LASTW), jnp.float32),
            pltpu.SemaphoreType.DMA,
            pltpu.SemaphoreType.DMA,
        ],
    )

    run_b = pl.kernel(
        _dot_body,
        out_type=jax.ShapeDtypeStruct((BATCH,), jnp.float32),
        mesh=mesh,
        compiler_params=pltpu.CompilerParams(
            needs_layout_passes=False, use_tc_tiling_on_sc=True),
        scratch_types=[
            pltpu.VMEM((_BPW // 2, EXT), jnp.float32),
            pltpu.VMEM((_BPW // 2, EXT), jnp.float32),
            pltpu.VMEM((_BPW,), jnp.float32),
            pltpu.VMEM((_L * _L,), jnp.float32),
            pltpu.SemaphoreType.DMA,
            pltpu.SemaphoreType.DMA,
        ],
    )

    iot = jnp.arange(BATCH, dtype=jnp.int32)
    su, pu = lax.sort_key_val(user, iot)
    sm, pm = lax.sort_key_val(movie, iot)

    urows, mrows = run_a(su >> 7, su & 127, pu, sm >> 7, sm & 127, pm,
                         user_factors.T, movie_factors.T,
                         user_biases.T, movie_biases.T)
    return run_b(urows, mrows)
